# Initial kernel scaffold; baseline (speedup 1.0000x reference)
import jax
import jax.numpy as jnp
from jax.experimental import pallas as pl


def kernel(x, edge_index, prottrans_feat, esm2, Wres, bres, W1, b1, g1, be1, W2, b2, g2, be2, Wf1, bf1, Wf2, bf2, Wf3, bf3):
    raise NotImplementedError("write your pallas kernel here")


# speedup vs baseline: 15.8406x; 15.8406x over previous
import functools

import jax
import jax.numpy as jnp
from jax import lax
from jax.experimental import pallas as pl
from jax.experimental.pallas import tpu as pltpu
from jax.experimental.pallas import tpu_sc as plsc

N = 10000
NPAD = 10240
E = 320000
NC, NS, L = 2, 16, 16
NW = NC * NS
CH = 128
CHUNKS = 160
EPW = CHUNKS * CH
EPAD = EPW * NS
DEG_EPW = EPAD // NW
STRIPE = NPAD // NS

_mesh = plsc.VectorSubcoreMesh(
    core_axis_name="c", subcore_axis_name="s", num_cores=NC, num_subcores=NS)
_sc_params = pltpu.CompilerParams(needs_layout_passes=False,
                                  use_tc_tiling_on_sc=False)


def _deg_body(dst_hbm, out_hbm, dst_v, deg_v):
    cid = lax.axis_index("c")
    sid = lax.axis_index("s")
    wid = sid * NC + cid
    pltpu.sync_copy(dst_hbm.at[wid], dst_v)

    def zero(i, _):
        deg_v[pl.ds(i * L, L)] = jnp.zeros((L,), jnp.float32)
        return 0
    lax.fori_loop(0, NPAD // L, zero, 0)

    ones = jnp.ones((L,), jnp.float32)

    def body(i, _):
        idx = dst_v[pl.ds(i * L, L)]
        plsc.addupdate_scatter(deg_v, [idx], ones)
        return 0
    lax.fori_loop(0, DEG_EPW // L, body, 0)
    pltpu.sync_copy(deg_v, out_hbm.at[wid])


def _sc_degree(dst2):
    return pl.kernel(
        _deg_body,
        out_type=jax.ShapeDtypeStruct((NW, NPAD), jnp.float32),
        mesh=_mesh,
        compiler_params=_sc_params,
        scratch_types=[
            pltpu.VMEM((DEG_EPW,), jnp.int32),
            pltpu.VMEM((NPAD,), jnp.float32),
        ],
    )(dst2)


def _agg_body(hs_lo, hs_hi, src_hbm, dst_hbm, zeros_hbm, out_lo, out_hi,
              src_v, dst_v, rows0, rows1, acc, sem0, sem1):
    cid = lax.axis_index("c")
    sid = lax.axis_index("s")
    pltpu.sync_copy(src_hbm.at[sid], src_v)
    pltpu.sync_copy(dst_hbm.at[sid], dst_v)

    def run_half(hs_ref, out_ref):
        pltpu.sync_copy(zeros_hbm.at[pl.ds(sid * STRIPE, STRIPE)],
                        acc.at[pl.ds(sid * STRIPE, STRIPE)])
        plsc.subcore_barrier()
        pltpu.async_copy(hs_ref.at[src_v.at[0]], rows0, sem0)

        def body(i, _):
            j0 = 2 * i
            d1 = pltpu.async_copy(hs_ref.at[src_v.at[j0 + 1]], rows1, sem1)
            pltpu.make_async_copy(hs_ref.at[src_v.at[0]], rows0, sem0).wait()
            pltpu.sync_copy(rows0, acc.at[dst_v.at[j0]], add=True)

            @pl.when(i < CHUNKS // 2 - 1)
            def _():
                pltpu.async_copy(hs_ref.at[src_v.at[j0 + 2]], rows0, sem0)
            d1.wait()
            pltpu.sync_copy(rows1, acc.at[dst_v.at[j0 + 1]], add=True)
            return 0
        lax.fori_loop(0, CHUNKS // 2, body, 0)

        plsc.subcore_barrier()
        pltpu.sync_copy(acc.at[pl.ds(sid * STRIPE, STRIPE)],
                        out_ref.at[pl.ds(sid * STRIPE, STRIPE)])

    @pl.when(cid == 0)
    def _():
        run_half(hs_lo, out_lo)

    @pl.when(cid == 1)
    def _():
        run_half(hs_hi, out_hi)


def _sc_aggregate(hs_lo, hs_hi, src3, dst3, zeros, dh):
    out = jax.ShapeDtypeStruct((NPAD, dh), jnp.float32)
    return pl.kernel(
        _agg_body,
        out_type=(out, out),
        mesh=_mesh,
        compiler_params=_sc_params,
        scratch_types=[
            pltpu.VMEM((CHUNKS, CH), jnp.int32),
            pltpu.VMEM((CHUNKS, CH), jnp.int32),
            pltpu.VMEM((CH, dh), jnp.float32),
            pltpu.VMEM((CH, dh), jnp.float32),
            pltpu.VMEM_SHARED((NPAD, dh), jnp.float32),
            pltpu.SemaphoreType.DMA,
            pltpu.SemaphoreType.DMA,
        ],
    )(hs_lo, hs_hi, src3, dst3, zeros)


def _dott(a, b):
    return lax.dot_general(a, b, (((1,), (1,)), ((), ())),
                           preferred_element_type=jnp.float32)


def _mm1_body(x_ref, w1_ref, wres_ref, bres_ref, h1_ref, xres_ref):
    xb = x_ref[...]
    h1_ref[...] = _dott(xb, w1_ref[...])
    xres_ref[...] = _dott(xb, wres_ref[...]) + bres_ref[...]


def _tc_mm1(x_pad, W1, Wres, bres_r):
    blk = NPAD // 10
    return pl.pallas_call(
        _mm1_body,
        grid=(10,),
        in_specs=[
            pl.BlockSpec((blk, 128), lambda i: (i, 0)),
            pl.BlockSpec((128, 128), lambda i: (0, 0)),
            pl.BlockSpec((128, 128), lambda i: (0, 0)),
            pl.BlockSpec((1, 128), lambda i: (0, 0)),
        ],
        out_specs=[
            pl.BlockSpec((blk, 128), lambda i: (i, 0)),
            pl.BlockSpec((blk, 128), lambda i: (i, 0)),
        ],
        out_shape=[
            jax.ShapeDtypeStruct((NPAD, 128), jnp.float32),
            jax.ShapeDtypeStruct((NPAD, 128), jnp.float32),
        ],
    )(x_pad, W1, Wres, bres_r)


def _fcpre_body(prot_ref, esm_ref, wb_ref, wc_ref, out_ref):
    out_ref[...] = (_dott(prot_ref[...], wb_ref[...]) +
                    _dott(esm_ref[...], wc_ref[...]))


def _tc_fcpre(prot, esm, Wf1b, Wf1c):
    blk = N // 10
    return pl.pallas_call(
        _fcpre_body,
        grid=(10,),
        in_specs=[
            pl.BlockSpec((blk, 1024), lambda i: (i, 0)),
            pl.BlockSpec((blk, 1280), lambda i: (i, 0)),
            pl.BlockSpec((512, 1024), lambda i: (0, 0)),
            pl.BlockSpec((512, 1280), lambda i: (0, 0)),
        ],
        out_specs=pl.BlockSpec((blk, 512), lambda i: (i, 0)),
        out_shape=jax.ShapeDtypeStruct((N, 512), jnp.float32),
    )(prot, esm, Wf1b, Wf1c)


def _scale_body(degp_ref, h1_ref, lo_ref, hi_ref, dinv_ref):
    degsum = jnp.sum(degp_ref[...], axis=0)
    dinv = lax.rsqrt(degsum + 1.0)
    dinv_ref[...] = dinv[:, None]
    hs = h1_ref[...] * dinv[:, None]
    lo_ref[...] = hs[:, :64]
    hi_ref[...] = hs[:, 64:]


def _tc_scale(deg_parts, h1):
    blk = NPAD // 10
    return pl.pallas_call(
        _scale_body,
        grid=(10,),
        in_specs=[
            pl.BlockSpec((NW, blk), lambda i: (0, i)),
            pl.BlockSpec((blk, 128), lambda i: (i, 0)),
        ],
        out_specs=[
            pl.BlockSpec((blk, 64), lambda i: (i, 0)),
            pl.BlockSpec((blk, 64), lambda i: (i, 0)),
            pl.BlockSpec((blk, 1), lambda i: (i, 0)),
        ],
        out_shape=[
            jax.ShapeDtypeStruct((NPAD, 64), jnp.float32),
            jax.ShapeDtypeStruct((NPAD, 64), jnp.float32),
            jax.ShapeDtypeStruct((NPAD, 1), jnp.float32),
        ],
    )(deg_parts, h1)


def _stats_body(plo_ref, phi_ref, hlo_ref, hhi_ref, dinv_ref, b_ref,
                t_ref, st_ref, *, d):
    i = pl.program_id(0)
    agg = jnp.concatenate([plo_ref[...] + hlo_ref[...],
                           phi_ref[...] + hhi_ref[...]], axis=1)
    t = agg * dinv_ref[...] + b_ref[...]
    t_ref[...] = t

    @pl.when(i == 0)
    def _():
        st_ref[...] = jnp.zeros((8, d), jnp.float32)
    blk = NPAD // 10
    rows = lax.broadcasted_iota(jnp.int32, (blk, 1), 0) + i * blk
    m = jnp.where(rows < N, 1.0, 0.0).astype(jnp.float32)
    tm = t * m
    s = jnp.sum(tm, axis=0)
    s2 = jnp.sum(tm * tm, axis=0)
    upd = jnp.concatenate(
        [s[None], s2[None], jnp.zeros((6, d), jnp.float32)], axis=0)
    st_ref[...] = st_ref[...] + upd


def _tc_stats(plo, phi, hlo, hhi, dinv, b_r, d):
    blk = NPAD // 10
    dh = d // 2
    return pl.pallas_call(
        functools.partial(_stats_body, d=d),
        grid=(10,),
        in_specs=[
            pl.BlockSpec((blk, dh), lambda i: (i, 0)),
            pl.BlockSpec((blk, dh), lambda i: (i, 0)),
            pl.BlockSpec((blk, dh), lambda i: (i, 0)),
            pl.BlockSpec((blk, dh), lambda i: (i, 0)),
            pl.BlockSpec((blk, 1), lambda i: (i, 0)),
            pl.BlockSpec((1, d), lambda i: (0, 0)),
        ],
        out_specs=[
            pl.BlockSpec((blk, d), lambda i: (i, 0)),
            pl.BlockSpec((8, d), lambda i: (0, 0)),
        ],
        out_shape=[
            jax.ShapeDtypeStruct((NPAD, d), jnp.float32),
            jax.ShapeDtypeStruct((8, d), jnp.float32),
        ],
    )(plo, phi, hlo, hhi, dinv, b_r)


def _mid_body(t1_ref, st_ref, g_ref, be_ref, xres_ref, w2_ref, dinv_ref,
              lo_ref, hi_ref):
    st = st_ref[...]
    mean = st[0] * (1.0 / N)
    var = st[1] * (1.0 / N) - mean * mean
    scale = g_ref[0] * lax.rsqrt(var + 1e-5)
    shift = be_ref[0] - mean * scale
    y = jnp.maximum(t1_ref[...] * scale + shift, 0.0) + xres_ref[...]
    h2 = _dott(y, w2_ref[...])
    hs2 = h2 * dinv_ref[...]
    lo_ref[...] = hs2[:, :32]
    hi_ref[...] = hs2[:, 32:]


def _tc_mid(t1, stats1, g1_r, be1_r, xres, W2, dinv):
    blk = NPAD // 10
    return pl.pallas_call(
        _mid_body,
        grid=(10,),
        in_specs=[
            pl.BlockSpec((blk, 128), lambda i: (i, 0)),
            pl.BlockSpec((8, 128), lambda i: (0, 0)),
            pl.BlockSpec((1, 128), lambda i: (0, 0)),
            pl.BlockSpec((1, 128), lambda i: (0, 0)),
            pl.BlockSpec((blk, 128), lambda i: (i, 0)),
            pl.BlockSpec((64, 128), lambda i: (0, 0)),
            pl.BlockSpec((blk, 1), lambda i: (i, 0)),
        ],
        out_specs=[
            pl.BlockSpec((blk, 32), lambda i: (i, 0)),
            pl.BlockSpec((blk, 32), lambda i: (i, 0)),
        ],
        out_shape=[
            jax.ShapeDtypeStruct((NPAD, 32), jnp.float32),
            jax.ShapeDtypeStruct((NPAD, 32), jnp.float32),
        ],
    )(t1, stats1, g1_r, be1_r, xres, W2, dinv)


def _fc_body(t2_ref, st_ref, g_ref, be_ref, pre_ref, wa_ref, bf1_ref,
             wf2_ref, bf2_ref, wf3_ref, bf3_ref, out_ref):
    st = st_ref[...]
    mean = st[0] * (1.0 / N)
    var = st[1] * (1.0 / N) - mean * mean
    scale = g_ref[0] * lax.rsqrt(var + 1e-5)
    shift = be_ref[0] - mean * scale
    y2 = jnp.maximum(t2_ref[...] * scale + shift, 0.0)
    a1 = jnp.maximum(_dott(y2, wa_ref[...]) + pre_ref[...] + bf1_ref[...],
                     0.0)
    a2 = jnp.maximum(_dott(a1, wf2_ref[...]) + bf2_ref[...], 0.0)
    out_ref[...] = _dott(a2, wf3_ref[...]) + bf3_ref[...]


def _tc_fc(t2, stats2, g2_r, be2_r, fcpre, Wf1a, bf1_r, Wf2, bf2_r,
           Wf3p, bf3_r):
    blk = N // 10
    return pl.pallas_call(
        _fc_body,
        grid=(10,),
        in_specs=[
            pl.BlockSpec((blk, 64), lambda i: (i, 0)),
            pl.BlockSpec((8, 64), lambda i: (0, 0)),
            pl.BlockSpec((1, 64), lambda i: (0, 0)),
            pl.BlockSpec((1, 64), lambda i: (0, 0)),
            pl.BlockSpec((blk, 512), lambda i: (i, 0)),
            pl.BlockSpec((512, 64), lambda i: (0, 0)),
            pl.BlockSpec((1, 512), lambda i: (0, 0)),
            pl.BlockSpec((128, 512), lambda i: (0, 0)),
            pl.BlockSpec((1, 128), lambda i: (0, 0)),
            pl.BlockSpec((128, 128), lambda i: (0, 0)),
            pl.BlockSpec((1, 128), lambda i: (0, 0)),
        ],
        out_specs=pl.BlockSpec((blk, 128), lambda i: (i, 0)),
        out_shape=jax.ShapeDtypeStruct((N, 128), jnp.float32),
    )(t2, stats2, g2_r, be2_r, fcpre, Wf1a, bf1_r, Wf2, bf2_r, Wf3p, bf3_r)


def kernel(x, edge_index, prottrans_feat, esm2,
           Wres, bres, W1, b1, g1, be1, W2, b2, g2, be2,
           Wf1, bf1, Wf2, bf2, Wf3, bf3):
    f32 = jnp.float32
    x_pad = jnp.pad(x, ((0, NPAD - N), (0, 0)))
    padv = jnp.full((EPAD - E,), N, jnp.int32)
    src_flat = jnp.concatenate([edge_index[0], padv])
    dst_flat = jnp.concatenate([edge_index[1], padv])
    src3 = src_flat.reshape(NS, CHUNKS, CH)
    dst3 = dst_flat.reshape(NS, CHUNKS, CH)
    dst2 = dst_flat.reshape(NW, DEG_EPW)
    zeros64 = jnp.zeros((NPAD, 64), f32)
    zeros32 = jnp.zeros((NPAD, 32), f32)

    bres_r = bres.reshape(1, 128)
    b1_r = b1.reshape(1, 128)
    g1_r = g1.reshape(1, 128)
    be1_r = be1.reshape(1, 128)
    b2_r = b2.reshape(1, 64)
    g2_r = g2.reshape(1, 64)
    be2_r = be2.reshape(1, 64)
    bf1_r = bf1.reshape(1, 512)
    bf2_r = bf2.reshape(1, 128)
    Wf1a = Wf1[:, :64]
    Wf1b = Wf1[:, 64:64 + 1024]
    Wf1c = Wf1[:, 64 + 1024:]
    Wf3p = jnp.zeros((128, 128), f32).at[:2].set(Wf3)
    bf3_r = jnp.zeros((1, 128), f32).at[0, :2].set(bf3)

    deg_parts = _sc_degree(dst2)
    h1, xres = _tc_mm1(x_pad, W1, Wres, bres_r)
    fcpre = _tc_fcpre(prottrans_feat, esm2, Wf1b, Wf1c)
    hs1_lo, hs1_hi, dinv = _tc_scale(deg_parts, h1)
    p1_lo, p1_hi = _sc_aggregate(hs1_lo, hs1_hi, src3, dst3, zeros64, 64)
    t1, stats1 = _tc_stats(p1_lo, p1_hi, hs1_lo, hs1_hi, dinv, b1_r, 128)
    hs2_lo, hs2_hi = _tc_mid(t1, stats1, g1_r, be1_r, xres, W2, dinv)
    p2_lo, p2_hi = _sc_aggregate(hs2_lo, hs2_hi, src3, dst3, zeros32, 32)
    t2, stats2 = _tc_stats(p2_lo, p2_hi, hs2_lo, hs2_hi, dinv, b2_r, 64)
    out128 = _tc_fc(t2, stats2, g2_r, be2_r, fcpre, Wf1a, bf1_r, Wf2,
                    bf2_r, Wf3p, bf3_r)
    return out128[:, :2]


# speedup vs baseline: 16.1767x; 1.0212x over previous
import functools

import jax
import jax.numpy as jnp
from jax import lax
from jax.experimental import pallas as pl
from jax.experimental.pallas import tpu as pltpu
from jax.experimental.pallas import tpu_sc as plsc

N = 10000
NPAD = 10240
E = 320000
NC, NS, L = 2, 16, 16
NW = NC * NS
CH = 128
CHUNKS = 160
EPW = CHUNKS * CH
EPAD = EPW * NS
DEG_EPW = EPAD // NW
STRIPE = NPAD // NS

_mesh = plsc.VectorSubcoreMesh(
    core_axis_name="c", subcore_axis_name="s", num_cores=NC, num_subcores=NS)
_sc_params = pltpu.CompilerParams(needs_layout_passes=False,
                                  use_tc_tiling_on_sc=False)


def _deg_body(dst_hbm, out_hbm, dst_v, deg_v):
    cid = lax.axis_index("c")
    sid = lax.axis_index("s")
    wid = sid * NC + cid
    pltpu.sync_copy(dst_hbm.at[wid], dst_v)

    def zero(i, _):
        deg_v[pl.ds(i * L, L)] = jnp.zeros((L,), jnp.float32)
        return 0
    lax.fori_loop(0, NPAD // L, zero, 0)

    ones = jnp.ones((L,), jnp.float32)

    def body(i, _):
        idx = dst_v[pl.ds(i * L, L)]
        plsc.addupdate_scatter(deg_v, [idx], ones)
        return 0
    lax.fori_loop(0, DEG_EPW // L, body, 0)
    pltpu.sync_copy(deg_v, out_hbm.at[wid])


def _sc_degree(dst2):
    return pl.kernel(
        _deg_body,
        out_type=jax.ShapeDtypeStruct((NW, NPAD), jnp.float32),
        mesh=_mesh,
        compiler_params=_sc_params,
        scratch_types=[
            pltpu.VMEM((DEG_EPW,), jnp.int32),
            pltpu.VMEM((NPAD,), jnp.float32),
        ],
    )(dst2)


def _agg_body(hs_lo, hs_hi, src_hbm, dst_hbm, zeros_hbm, out_lo, out_hi,
              src_v, dst_v, rows0, rows1, acc, sem0, sem1):
    cid = lax.axis_index("c")
    sid = lax.axis_index("s")
    pltpu.sync_copy(src_hbm.at[sid], src_v)
    pltpu.sync_copy(dst_hbm.at[sid], dst_v)

    def run_half(hs_ref, out_ref):
        pltpu.sync_copy(zeros_hbm.at[pl.ds(sid * STRIPE, STRIPE)],
                        acc.at[pl.ds(sid * STRIPE, STRIPE)])
        plsc.subcore_barrier()
        pltpu.async_copy(hs_ref.at[src_v.at[0]], rows0, sem0)

        def body(i, _):
            j0 = 2 * i
            d1 = pltpu.async_copy(hs_ref.at[src_v.at[j0 + 1]], rows1, sem1)
            pltpu.make_async_copy(hs_ref.at[src_v.at[0]], rows0, sem0).wait()
            pltpu.sync_copy(rows0, acc.at[dst_v.at[j0]], add=True)

            @pl.when(i < CHUNKS // 2 - 1)
            def _():
                pltpu.async_copy(hs_ref.at[src_v.at[j0 + 2]], rows0, sem0)
            d1.wait()
            pltpu.sync_copy(rows1, acc.at[dst_v.at[j0 + 1]], add=True)
            return 0
        lax.fori_loop(0, CHUNKS // 2, body, 0)

        plsc.subcore_barrier()
        pltpu.sync_copy(acc.at[pl.ds(sid * STRIPE, STRIPE)],
                        out_ref.at[pl.ds(sid * STRIPE, STRIPE)])

    @pl.when(cid == 0)
    def _():
        run_half(hs_lo, out_lo)

    @pl.when(cid == 1)
    def _():
        run_half(hs_hi, out_hi)


def _sc_aggregate(hs_lo, hs_hi, src3, dst3, zeros, dh):
    out = jax.ShapeDtypeStruct((NPAD, dh), jnp.float32)
    return pl.kernel(
        _agg_body,
        out_type=(out, out),
        mesh=_mesh,
        compiler_params=_sc_params,
        scratch_types=[
            pltpu.VMEM((CHUNKS, CH), jnp.int32),
            pltpu.VMEM((CHUNKS, CH), jnp.int32),
            pltpu.VMEM((CH, dh), jnp.float32),
            pltpu.VMEM((CH, dh), jnp.float32),
            pltpu.VMEM_SHARED((NPAD, dh), jnp.float32),
            pltpu.SemaphoreType.DMA,
            pltpu.SemaphoreType.DMA,
        ],
    )(hs_lo, hs_hi, src3, dst3, zeros)


def _dott(a, b):
    return lax.dot_general(a, b, (((1,), (1,)), ((), ())),
                           preferred_element_type=jnp.float32)


def _mm1_body(x_ref, w1_ref, wres_ref, bres_ref, h1_ref, xres_ref):
    xb = x_ref[...]
    h1_ref[...] = _dott(xb, w1_ref[...])
    xres_ref[...] = _dott(xb, wres_ref[...]) + bres_ref[...]


def _tc_mm1(x_pad, W1, Wres, bres_r):
    blk = NPAD // 10
    return pl.pallas_call(
        _mm1_body,
        grid=(10,),
        in_specs=[
            pl.BlockSpec((blk, 128), lambda i: (i, 0)),
            pl.BlockSpec((128, 128), lambda i: (0, 0)),
            pl.BlockSpec((128, 128), lambda i: (0, 0)),
            pl.BlockSpec((1, 128), lambda i: (0, 0)),
        ],
        out_specs=[
            pl.BlockSpec((blk, 128), lambda i: (i, 0)),
            pl.BlockSpec((blk, 128), lambda i: (i, 0)),
        ],
        out_shape=[
            jax.ShapeDtypeStruct((NPAD, 128), jnp.float32),
            jax.ShapeDtypeStruct((NPAD, 128), jnp.float32),
        ],
    )(x_pad, W1, Wres, bres_r)


def _fcpre_body(prot_ref, esm_ref, wb_ref, wc_ref, out_ref):
    out_ref[...] = (_dott(prot_ref[...], wb_ref[...]) +
                    _dott(esm_ref[...], wc_ref[...]))


def _tc_fcpre(prot, esm, Wf1b, Wf1c):
    blk = N // 10
    return pl.pallas_call(
        _fcpre_body,
        grid=(10,),
        in_specs=[
            pl.BlockSpec((blk, 1024), lambda i: (i, 0)),
            pl.BlockSpec((blk, 1280), lambda i: (i, 0)),
            pl.BlockSpec((512, 1024), lambda i: (0, 0)),
            pl.BlockSpec((512, 1280), lambda i: (0, 0)),
        ],
        out_specs=pl.BlockSpec((blk, 512), lambda i: (i, 0)),
        out_shape=jax.ShapeDtypeStruct((N, 512), jnp.float32),
    )(prot, esm, Wf1b, Wf1c)


def _scale_body(degp_ref, h1_ref, lo_ref, hi_ref, dinv_ref):
    degsum = jnp.sum(degp_ref[...], axis=0)
    dinv = lax.rsqrt(degsum + 1.0)
    dinv_ref[...] = dinv[:, None]
    hs = h1_ref[...] * dinv[:, None]
    lo_ref[...] = hs[:, :64]
    hi_ref[...] = hs[:, 64:]


def _tc_scale(deg_parts, h1):
    blk = NPAD // 10
    return pl.pallas_call(
        _scale_body,
        grid=(10,),
        in_specs=[
            pl.BlockSpec((NW, blk), lambda i: (0, i)),
            pl.BlockSpec((blk, 128), lambda i: (i, 0)),
        ],
        out_specs=[
            pl.BlockSpec((blk, 64), lambda i: (i, 0)),
            pl.BlockSpec((blk, 64), lambda i: (i, 0)),
            pl.BlockSpec((blk, 1), lambda i: (i, 0)),
        ],
        out_shape=[
            jax.ShapeDtypeStruct((NPAD, 64), jnp.float32),
            jax.ShapeDtypeStruct((NPAD, 64), jnp.float32),
            jax.ShapeDtypeStruct((NPAD, 1), jnp.float32),
        ],
    )(deg_parts, h1)


def _stats_body(plo_ref, phi_ref, hlo_ref, hhi_ref, dinv_ref, b_ref,
                t_ref, st_ref, *, d):
    i = pl.program_id(0)
    agg = jnp.concatenate([plo_ref[...] + hlo_ref[...],
                           phi_ref[...] + hhi_ref[...]], axis=1)
    t = agg * dinv_ref[...] + b_ref[...]
    t_ref[...] = t

    @pl.when(i == 0)
    def _():
        st_ref[...] = jnp.zeros((8, d), jnp.float32)
    blk = NPAD // 10
    rows = lax.broadcasted_iota(jnp.int32, (blk, 1), 0) + i * blk
    m = jnp.where(rows < N, 1.0, 0.0).astype(jnp.float32)
    tm = t * m
    s = jnp.sum(tm, axis=0)
    s2 = jnp.sum(tm * tm, axis=0)
    upd = jnp.concatenate(
        [s[None], s2[None], jnp.zeros((6, d), jnp.float32)], axis=0)
    st_ref[...] = st_ref[...] + upd


def _tc_stats(plo, phi, hlo, hhi, dinv, b_r, d):
    blk = NPAD // 10
    dh = d // 2
    return pl.pallas_call(
        functools.partial(_stats_body, d=d),
        grid=(10,),
        in_specs=[
            pl.BlockSpec((blk, dh), lambda i: (i, 0)),
            pl.BlockSpec((blk, dh), lambda i: (i, 0)),
            pl.BlockSpec((blk, dh), lambda i: (i, 0)),
            pl.BlockSpec((blk, dh), lambda i: (i, 0)),
            pl.BlockSpec((blk, 1), lambda i: (i, 0)),
            pl.BlockSpec((1, d), lambda i: (0, 0)),
        ],
        out_specs=[
            pl.BlockSpec((blk, d), lambda i: (i, 0)),
            pl.BlockSpec((8, d), lambda i: (0, 0)),
        ],
        out_shape=[
            jax.ShapeDtypeStruct((NPAD, d), jnp.float32),
            jax.ShapeDtypeStruct((8, d), jnp.float32),
        ],
    )(plo, phi, hlo, hhi, dinv, b_r)


def _mid_body(t1_ref, st_ref, g_ref, be_ref, xres_ref, w2_ref, dinv_ref,
              lo_ref, hi_ref):
    st = st_ref[...]
    mean = st[0] * (1.0 / N)
    var = st[1] * (1.0 / N) - mean * mean
    scale = g_ref[0] * lax.rsqrt(var + 1e-5)
    shift = be_ref[0] - mean * scale
    y = jnp.maximum(t1_ref[...] * scale + shift, 0.0) + xres_ref[...]
    h2 = _dott(y, w2_ref[...])
    hs2 = h2 * dinv_ref[...]
    lo_ref[...] = hs2[:, :32]
    hi_ref[...] = hs2[:, 32:]


def _tc_mid(t1, stats1, g1_r, be1_r, xres, W2, dinv):
    blk = NPAD // 10
    return pl.pallas_call(
        _mid_body,
        grid=(10,),
        in_specs=[
            pl.BlockSpec((blk, 128), lambda i: (i, 0)),
            pl.BlockSpec((8, 128), lambda i: (0, 0)),
            pl.BlockSpec((1, 128), lambda i: (0, 0)),
            pl.BlockSpec((1, 128), lambda i: (0, 0)),
            pl.BlockSpec((blk, 128), lambda i: (i, 0)),
            pl.BlockSpec((64, 128), lambda i: (0, 0)),
            pl.BlockSpec((blk, 1), lambda i: (i, 0)),
        ],
        out_specs=[
            pl.BlockSpec((blk, 32), lambda i: (i, 0)),
            pl.BlockSpec((blk, 32), lambda i: (i, 0)),
        ],
        out_shape=[
            jax.ShapeDtypeStruct((NPAD, 32), jnp.float32),
            jax.ShapeDtypeStruct((NPAD, 32), jnp.float32),
        ],
    )(t1, stats1, g1_r, be1_r, xres, W2, dinv)


def _fc_body(t2_ref, st_ref, g_ref, be_ref, pre_ref, wa_ref, bf1_ref,
             wf2_ref, bf2_ref, wf3_ref, bf3_ref, out_ref):
    st = st_ref[...]
    mean = st[0] * (1.0 / N)
    var = st[1] * (1.0 / N) - mean * mean
    scale = g_ref[0] * lax.rsqrt(var + 1e-5)
    shift = be_ref[0] - mean * scale
    y2 = jnp.maximum(t2_ref[...] * scale + shift, 0.0)
    a1 = jnp.maximum(_dott(y2, wa_ref[...]) + pre_ref[...] + bf1_ref[...],
                     0.0)
    a2 = jnp.maximum(_dott(a1, wf2_ref[...]) + bf2_ref[...], 0.0)
    out_ref[...] = _dott(a2, wf3_ref[...]) + bf3_ref[...]


def _tc_fc(t2, stats2, g2_r, be2_r, fcpre, Wf1a, bf1_r, Wf2, bf2_r,
           Wf3p, bf3_r):
    blk = N // 10
    return pl.pallas_call(
        _fc_body,
        grid=(10,),
        in_specs=[
            pl.BlockSpec((blk, 64), lambda i: (i, 0)),
            pl.BlockSpec((8, 64), lambda i: (0, 0)),
            pl.BlockSpec((1, 64), lambda i: (0, 0)),
            pl.BlockSpec((1, 64), lambda i: (0, 0)),
            pl.BlockSpec((blk, 512), lambda i: (i, 0)),
            pl.BlockSpec((512, 64), lambda i: (0, 0)),
            pl.BlockSpec((1, 512), lambda i: (0, 0)),
            pl.BlockSpec((128, 512), lambda i: (0, 0)),
            pl.BlockSpec((1, 128), lambda i: (0, 0)),
            pl.BlockSpec((128, 128), lambda i: (0, 0)),
            pl.BlockSpec((1, 128), lambda i: (0, 0)),
        ],
        out_specs=pl.BlockSpec((blk, 128), lambda i: (i, 0)),
        out_shape=jax.ShapeDtypeStruct((N, 128), jnp.float32),
    )(t2, stats2, g2_r, be2_r, fcpre, Wf1a, bf1_r, Wf2, bf2_r, Wf3p, bf3_r)


def kernel(x, edge_index, prottrans_feat, esm2,
           Wres, bres, W1, b1, g1, be1, W2, b2, g2, be2,
           Wf1, bf1, Wf2, bf2, Wf3, bf3):
    f32 = jnp.float32
    x_pad = jnp.pad(x, ((0, NPAD - N), (0, 0)))
    padv = jnp.full((EPAD - E,), N, jnp.int32)
    src_flat = jnp.concatenate([edge_index[0], padv])
    dst_flat = jnp.concatenate([edge_index[1], padv])
    src3 = src_flat.reshape(NS, CHUNKS, CH)
    dst3 = dst_flat.reshape(NS, CHUNKS, CH)
    dst2 = dst_flat.reshape(NW, DEG_EPW)
    zeros64 = jnp.zeros((NPAD, 64), f32)
    zeros32 = jnp.zeros((NPAD, 32), f32)

    bres_r = bres.reshape(1, 128)
    b1_r = b1.reshape(1, 128)
    g1_r = g1.reshape(1, 128)
    be1_r = be1.reshape(1, 128)
    b2_r = b2.reshape(1, 64)
    g2_r = g2.reshape(1, 64)
    be2_r = be2.reshape(1, 64)
    bf1_r = bf1.reshape(1, 512)
    bf2_r = bf2.reshape(1, 128)
    Wf1a = Wf1[:, :64]
    Wf1b = Wf1[:, 64:64 + 1024]
    Wf1c = Wf1[:, 64 + 1024:]
    Wf3p = jnp.zeros((128, 128), f32).at[:2].set(Wf3)
    bf3_r = jnp.zeros((1, 128), f32).at[0, :2].set(bf3)

    deg_parts = _sc_degree(dst2)
    h1, xres = _tc_mm1(x_pad, W1, Wres, bres_r)
    hs1_lo, hs1_hi, dinv = _tc_scale(deg_parts, h1)
    p1_lo, p1_hi = _sc_aggregate(hs1_lo, hs1_hi, src3, dst3, zeros64, 64)
    prot_d, esm_d, _ = lax.optimization_barrier(
        (prottrans_feat, esm2, hs1_lo))
    fcpre = _tc_fcpre(prot_d, esm_d, Wf1b, Wf1c)
    t1, stats1 = _tc_stats(p1_lo, p1_hi, hs1_lo, hs1_hi, dinv, b1_r, 128)
    hs2_lo, hs2_hi = _tc_mid(t1, stats1, g1_r, be1_r, xres, W2, dinv)
    p2_lo, p2_hi = _sc_aggregate(hs2_lo, hs2_hi, src3, dst3, zeros32, 32)
    t2, stats2 = _tc_stats(p2_lo, p2_hi, hs2_lo, hs2_hi, dinv, b2_r, 64)
    out128 = _tc_fc(t2, stats2, g2_r, be2_r, fcpre, Wf1a, bf1_r, Wf2,
                    bf2_r, Wf3p, bf3_r)
    return out128[:, :2]
